# baseline (speedup 1.0000x reference)
import functools

import jax
import jax.numpy as jnp
from jax import lax
from jax.experimental import pallas as pl
from jax.experimental.pallas import tpu as pltpu
from jax.experimental.pallas import tpu_sc as plsc

N_NODES = 100000
N_EDGES = 6400000
LANES = 128
KB = 125
R = N_EDGES // LANES
NB = R // KB
NC, NS = 2, 16
NW = NC * NS
ACC_PAD = 100096
SLICE = ACC_PAD // NS
ST_CHUNK = 4000

_mesh = plsc.VectorSubcoreMesh(core_axis_name="c", subcore_axis_name="s")


@functools.partial(
    pl.kernel,
    out_type=jax.ShapeDtypeStruct((NC * ACC_PAD,), jnp.float32),
    mesh=_mesh,
    scratch_types=[
        pltpu.VMEM((2, KB * LANES), jnp.int32),
        pltpu.VMEM((2, KB * LANES), jnp.int32),
        pltpu.VMEM((2, KB, LANES), jnp.float32),
        pltpu.VMEM((SLICE,), jnp.float32),
        pltpu.VMEM_SHARED((N_NODES,), jnp.float32),
        pltpu.VMEM_SHARED((ACC_PAD,), jnp.float32),
        pltpu.SemaphoreType.DMA,
        pltpu.SemaphoreType.DMA,
        pltpu.SemaphoreType.DMA,
        pltpu.SemaphoreType.DMA,
        pltpu.SemaphoreType.DMA,
    ],
)
def _scatter_kernel(ei, state, part, src_i, dst_i, vals, bounce,
                    state_sh, acc_sh, isem, gsem_a, gsem_b, ssem_a, ssem_b):
    c = lax.axis_index("c")
    s = lax.axis_index("s")
    w = s * NC + c

    def zbody(i, carry):
        bounce[pl.ds(i * 16, 16)] = jnp.zeros((16,), jnp.float32)
        return carry

    lax.fori_loop(0, SLICE // 16, zbody, 0)
    sl = pl.ds(s * SLICE, SLICE)
    pltpu.sync_copy(bounce, acc_sh.at[sl])

    st = pl.ds(s * ST_CHUNK, ST_CHUNK)
    bs = bounce.at[pl.ds(0, ST_CHUNK)]
    pltpu.sync_copy(state.at[st], bs)
    pltpu.sync_copy(bs, state_sh.at[st])

    @pl.when(s < N_NODES // ST_CHUNK - NS)
    def _():
        st2 = pl.ds((s + NS) * ST_CHUNK, ST_CHUNK)
        pltpu.sync_copy(state.at[st2], bs)
        pltpu.sync_copy(bs, state_sh.at[st2])

    plsc.subcore_barrier()

    nb = (NB - w + NW - 1) // NW
    gsems = (gsem_a, gsem_b)
    ssems = (ssem_a, ssem_b)

    def stage_and_fire(bi, buf):
        row0 = (w + bi * NW) * (KB * LANES)
        pltpu.async_copy(ei.at[0, pl.ds(row0, KB * LANES)], src_i.at[buf],
                         isem)
        pltpu.async_copy(ei.at[1, pl.ds(row0, KB * LANES)], dst_i.at[buf],
                         isem)
        pltpu.make_async_copy(ei.at[0, pl.ds(row0, KB * LANES)],
                              src_i.at[buf], isem).wait()
        pltpu.make_async_copy(ei.at[1, pl.ds(row0, KB * LANES)],
                              dst_i.at[buf], isem).wait()
        for j in range(KB):
            jc = pl.ds(j * LANES, LANES)
            pltpu.async_copy(state_sh.at[src_i.at[buf, jc]],
                             vals.at[buf, j], gsems[buf])

    def drain_gather(buf):
        for j in range(KB):
            jc = pl.ds(j * LANES, LANES)
            pltpu.make_async_copy(state_sh.at[src_i.at[buf, jc]],
                                  vals.at[buf, j], gsems[buf]).wait()

    def fire_scatter(buf):
        for j in range(KB):
            jc = pl.ds(j * LANES, LANES)
            pltpu.async_copy(vals.at[buf, j], acc_sh.at[dst_i.at[buf, jc]],
                             ssems[buf], add=True)

    def drain_scatter(buf):
        for j in range(KB):
            jc = pl.ds(j * LANES, LANES)
            pltpu.make_async_copy(vals.at[buf, j],
                                  acc_sh.at[dst_i.at[buf, jc]],
                                  ssems[buf]).wait()

    def step(bi, cur, nxt):
        @pl.when(bi >= 1)
        def _():
            drain_scatter(nxt)

        @pl.when(bi + 1 < nb)
        def _():
            stage_and_fire(bi + 1, nxt)

        drain_gather(cur)
        fire_scatter(cur)

    stage_and_fire(0, 0)

    def body(p, carry):
        bi = p * 2
        step(bi, 0, 1)
        step(bi + 1, 1, 0)
        return carry

    lax.fori_loop(0, nb // 2, body, 0)

    @pl.when(nb % 2 == 1)
    def _():
        step(nb - 1, 0, 1)

    @pl.when(nb % 2 == 1)
    def _():
        drain_scatter(0)

    @pl.when(nb % 2 == 0)
    def _():
        drain_scatter(1)

    plsc.subcore_barrier()

    pltpu.sync_copy(acc_sh.at[sl], bounce)
    pltpu.sync_copy(bounce, part.at[pl.ds(c * ACC_PAD + s * SLICE, SLICE)])


CB = 4000
NT_B = N_NODES // CB


@functools.partial(
    pl.kernel,
    out_type=jax.ShapeDtypeStruct((N_NODES,), jnp.float32),
    mesh=_mesh,
    scratch_types=[
        pltpu.VMEM((CB,), jnp.float32),
        pltpu.VMEM((CB,), jnp.float32),
        pltpu.VMEM((CB,), jnp.float32),
        pltpu.VMEM((CB,), jnp.float32),
        pltpu.VMEM((16,), jnp.float32),
        pltpu.VMEM((16,), jnp.float32),
        pltpu.VMEM((16,), jnp.float32),
    ],
)
def _combine_kernel(part, state, w0, w1, bb, out,
                    p0v, p1v, sv, ov, w0v, w1v, bbv):
    c = lax.axis_index("c")
    s = lax.axis_index("s")
    w = s * NC + c

    @pl.when(w < NT_B)
    def _():
        base = w * CB
        pltpu.sync_copy(part.at[pl.ds(base, CB)], p0v)
        pltpu.sync_copy(part.at[pl.ds(ACC_PAD + base, CB)], p1v)
        pltpu.sync_copy(state.at[pl.ds(base, CB)], sv)
        pltpu.sync_copy(w0, w0v)
        pltpu.sync_copy(w1, w1v)
        pltpu.sync_copy(bb, bbv)
        a0 = w0v[...]
        a1 = w1v[...]
        ab = bbv[...]

        def body(i, carry):
            ch = pl.ds(i * 16, 16)
            ov[ch] = sv[ch] * a0 + (p0v[ch] + p1v[ch]) * a1 + ab
            return carry

        lax.fori_loop(0, CB // 16, body, 0)
        pltpu.sync_copy(ov, out.at[pl.ds(base, CB)])


def kernel(state, edge_index, W, b):
    if state.ndim == 1:
        state = state[:, None]
    sflat = state.reshape(-1).astype(jnp.float32)
    ei = edge_index.astype(jnp.int32)
    part = _scatter_kernel(ei, sflat)
    w0 = jnp.full((16,), W[0, 0], jnp.float32)
    w1 = jnp.full((16,), W[0, 1], jnp.float32)
    bb = jnp.full((16,), b[0], jnp.float32)
    out = _combine_kernel(part, sflat, w0, w1, bb)
    return out.reshape(N_NODES, 1)


# speedup vs baseline: 1.0011x; 1.0011x over previous
import functools

import jax
import jax.numpy as jnp
from jax import lax
from jax.experimental import pallas as pl
from jax.experimental.pallas import tpu as pltpu
from jax.experimental.pallas import tpu_sc as plsc

N_NODES = 100000
N_EDGES = 6400000
LANES = 128
KB = 100
R = N_EDGES // LANES
NB = R // KB
NC, NS = 2, 16
NW = NC * NS
ACC_PAD = 100096
SLICE = ACC_PAD // NS
ST_CHUNK = 4000

_mesh = plsc.VectorSubcoreMesh(core_axis_name="c", subcore_axis_name="s")


@functools.partial(
    pl.kernel,
    out_type=jax.ShapeDtypeStruct((NC * ACC_PAD,), jnp.float32),
    mesh=_mesh,
    scratch_types=[
        pltpu.VMEM((2, KB * LANES), jnp.int32),
        pltpu.VMEM((2, KB * LANES), jnp.int32),
        pltpu.VMEM((2, KB, LANES), jnp.float32),
        pltpu.VMEM((SLICE,), jnp.float32),
        pltpu.VMEM_SHARED((N_NODES,), jnp.float32),
        pltpu.VMEM_SHARED((ACC_PAD,), jnp.float32),
        pltpu.SemaphoreType.DMA,
        pltpu.SemaphoreType.DMA,
        pltpu.SemaphoreType.DMA,
        pltpu.SemaphoreType.DMA,
        pltpu.SemaphoreType.DMA,
    ],
)
def _scatter_kernel(ei, state, part, src_i, dst_i, vals, bounce,
                    state_sh, acc_sh, isem, gsem_a, gsem_b, ssem_a, ssem_b):
    c = lax.axis_index("c")
    s = lax.axis_index("s")
    w = s * NC + c

    def zbody(i, carry):
        bounce[pl.ds(i * 16, 16)] = jnp.zeros((16,), jnp.float32)
        return carry

    lax.fori_loop(0, SLICE // 16, zbody, 0)
    sl = pl.ds(s * SLICE, SLICE)
    pltpu.sync_copy(bounce, acc_sh.at[sl])

    st = pl.ds(s * ST_CHUNK, ST_CHUNK)
    bs = bounce.at[pl.ds(0, ST_CHUNK)]
    pltpu.sync_copy(state.at[st], bs)
    pltpu.sync_copy(bs, state_sh.at[st])

    @pl.when(s < N_NODES // ST_CHUNK - NS)
    def _():
        st2 = pl.ds((s + NS) * ST_CHUNK, ST_CHUNK)
        pltpu.sync_copy(state.at[st2], bs)
        pltpu.sync_copy(bs, state_sh.at[st2])

    plsc.subcore_barrier()

    nb = (NB - w + NW - 1) // NW
    gsems = (gsem_a, gsem_b)
    ssems = (ssem_a, ssem_b)

    def stage_and_fire(bi, buf):
        row0 = (w + bi * NW) * (KB * LANES)
        pltpu.async_copy(ei.at[0, pl.ds(row0, KB * LANES)], src_i.at[buf],
                         isem)
        pltpu.async_copy(ei.at[1, pl.ds(row0, KB * LANES)], dst_i.at[buf],
                         isem)
        pltpu.make_async_copy(ei.at[0, pl.ds(row0, KB * LANES)],
                              src_i.at[buf], isem).wait()
        pltpu.make_async_copy(ei.at[1, pl.ds(row0, KB * LANES)],
                              dst_i.at[buf], isem).wait()
        for j in range(KB):
            jc = pl.ds(j * LANES, LANES)
            pltpu.async_copy(state_sh.at[src_i.at[buf, jc]],
                             vals.at[buf, j], gsems[buf])

    def drain_gather(buf):
        for j in range(KB):
            jc = pl.ds(j * LANES, LANES)
            pltpu.make_async_copy(state_sh.at[src_i.at[buf, jc]],
                                  vals.at[buf, j], gsems[buf]).wait()

    def fire_scatter(buf):
        for j in range(KB):
            jc = pl.ds(j * LANES, LANES)
            pltpu.async_copy(vals.at[buf, j], acc_sh.at[dst_i.at[buf, jc]],
                             ssems[buf], add=True)

    def drain_scatter(buf):
        for j in range(KB):
            jc = pl.ds(j * LANES, LANES)
            pltpu.make_async_copy(vals.at[buf, j],
                                  acc_sh.at[dst_i.at[buf, jc]],
                                  ssems[buf]).wait()

    def step(bi, cur, nxt):
        @pl.when(bi >= 1)
        def _():
            drain_scatter(nxt)

        @pl.when(bi + 1 < nb)
        def _():
            stage_and_fire(bi + 1, nxt)

        drain_gather(cur)
        fire_scatter(cur)

    stage_and_fire(0, 0)

    def body(p, carry):
        bi = p * 2
        step(bi, 0, 1)
        step(bi + 1, 1, 0)
        return carry

    lax.fori_loop(0, nb // 2, body, 0)

    @pl.when(nb % 2 == 1)
    def _():
        step(nb - 1, 0, 1)

    @pl.when(nb % 2 == 1)
    def _():
        drain_scatter(0)

    @pl.when(nb % 2 == 0)
    def _():
        drain_scatter(1)

    plsc.subcore_barrier()

    pltpu.sync_copy(acc_sh.at[sl], bounce)
    pltpu.sync_copy(bounce, part.at[pl.ds(c * ACC_PAD + s * SLICE, SLICE)])


CB = 4000
NT_B = N_NODES // CB


@functools.partial(
    pl.kernel,
    out_type=jax.ShapeDtypeStruct((N_NODES,), jnp.float32),
    mesh=_mesh,
    scratch_types=[
        pltpu.VMEM((CB,), jnp.float32),
        pltpu.VMEM((CB,), jnp.float32),
        pltpu.VMEM((CB,), jnp.float32),
        pltpu.VMEM((CB,), jnp.float32),
        pltpu.VMEM((16,), jnp.float32),
        pltpu.VMEM((16,), jnp.float32),
        pltpu.VMEM((16,), jnp.float32),
    ],
)
def _combine_kernel(part, state, w0, w1, bb, out,
                    p0v, p1v, sv, ov, w0v, w1v, bbv):
    c = lax.axis_index("c")
    s = lax.axis_index("s")
    w = s * NC + c

    @pl.when(w < NT_B)
    def _():
        base = w * CB
        pltpu.sync_copy(part.at[pl.ds(base, CB)], p0v)
        pltpu.sync_copy(part.at[pl.ds(ACC_PAD + base, CB)], p1v)
        pltpu.sync_copy(state.at[pl.ds(base, CB)], sv)
        pltpu.sync_copy(w0, w0v)
        pltpu.sync_copy(w1, w1v)
        pltpu.sync_copy(bb, bbv)
        a0 = w0v[...]
        a1 = w1v[...]
        ab = bbv[...]

        def body(i, carry):
            ch = pl.ds(i * 16, 16)
            ov[ch] = sv[ch] * a0 + (p0v[ch] + p1v[ch]) * a1 + ab
            return carry

        lax.fori_loop(0, CB // 16, body, 0)
        pltpu.sync_copy(ov, out.at[pl.ds(base, CB)])


def kernel(state, edge_index, W, b):
    if state.ndim == 1:
        state = state[:, None]
    sflat = state.reshape(-1).astype(jnp.float32)
    ei = edge_index.astype(jnp.int32)
    part = _scatter_kernel(ei, sflat)
    w0 = jnp.full((16,), W[0, 0], jnp.float32)
    w1 = jnp.full((16,), W[0, 1], jnp.float32)
    bb = jnp.full((16,), b[0], jnp.float32)
    out = _combine_kernel(part, sflat, w0, w1, bb)
    return out.reshape(N_NODES, 1)


# speedup vs baseline: 1.1630x; 1.1617x over previous
import functools

import jax
import jax.numpy as jnp
from jax import lax
from jax.experimental import pallas as pl
from jax.experimental.pallas import tpu as pltpu
from jax.experimental.pallas import tpu_sc as plsc

N_NODES = 100000
N_EDGES = 6400000
LANES = 128
KB = 16
R = N_EDGES // LANES
NB = R // KB
NC, NS = 2, 16
NW = NC * NS
ACC_PAD = 100096
SLICE = ACC_PAD // NS
ST_ROWS = ACC_PAD // LANES

_mesh = plsc.VectorSubcoreMesh(core_axis_name="c", subcore_axis_name="s")


@functools.partial(
    pl.kernel,
    out_type=jax.ShapeDtypeStruct((NC * ACC_PAD,), jnp.float32),
    mesh=_mesh,
    compiler_params=pltpu.CompilerParams(needs_layout_passes=False),
    scratch_types=[
        pltpu.VMEM((2, KB * LANES), jnp.int32),
        pltpu.VMEM((2, KB * LANES), jnp.int32),
        pltpu.VMEM((2, KB, LANES), jnp.float32),
        pltpu.VMEM((SLICE,), jnp.float32),
        pltpu.VMEM((ST_ROWS, LANES), jnp.float32),
        pltpu.VMEM_SHARED((ACC_PAD,), jnp.float32),
        pltpu.SemaphoreType.DMA,
        pltpu.SemaphoreType.DMA,
        pltpu.SemaphoreType.DMA,
        pltpu.SemaphoreType.DMA,
    ],
)
def _scatter_kernel(ei, state2d, part, src_i, dst_i, vals, bounce, state_t,
                    acc_sh, isem_a, isem_b, ssem_a, ssem_b):
    c = lax.axis_index("c")
    s = lax.axis_index("s")
    w = s * NC + c

    def zbody(i, carry):
        bounce[pl.ds(i * 16, 16)] = jnp.zeros((16,), jnp.float32)
        return carry

    lax.fori_loop(0, SLICE // 16, zbody, 0)
    sl = pl.ds(s * SLICE, SLICE)
    pltpu.sync_copy(bounce, acc_sh.at[sl])
    pltpu.sync_copy(state2d, state_t)
    plsc.subcore_barrier()

    nb = (NB - w + NW - 1) // NW
    isems = (isem_a, isem_b)
    ssems = (ssem_a, ssem_b)

    def stage_idx(bi, buf):
        row0 = (w + bi * NW) * (KB * LANES)
        pltpu.async_copy(ei.at[0, pl.ds(row0, KB * LANES)], src_i.at[buf],
                         isems[buf])
        pltpu.async_copy(ei.at[1, pl.ds(row0, KB * LANES)], dst_i.at[buf],
                         isems[buf])

    def wait_idx(buf):
        pltpu.make_async_copy(ei.at[0, pl.ds(0, KB * LANES)], src_i.at[buf],
                              isems[buf]).wait()
        pltpu.make_async_copy(ei.at[1, pl.ds(0, KB * LANES)], dst_i.at[buf],
                              isems[buf]).wait()

    def gather_rows(buf):
        for j in range(KB):
            for k in range(LANES // 16):
                iv = src_i[buf, pl.ds(j * LANES + k * 16, 16)]
                row = jax.lax.shift_right_logical(iv, 7)
                col = jax.lax.bitwise_and(iv, 127)
                vals[buf, j, pl.ds(k * 16, 16)] = plsc.load_gather(
                    state_t, [row, col])

    def fire_scatter(buf):
        for j in range(KB):
            jc = pl.ds(j * LANES, LANES)
            pltpu.async_copy(vals.at[buf, j], acc_sh.at[dst_i.at[buf, jc]],
                             ssems[buf], add=True)

    def drain_scatter(buf):
        for j in range(KB):
            jc = pl.ds(j * LANES, LANES)
            pltpu.make_async_copy(vals.at[buf, j],
                                  acc_sh.at[dst_i.at[buf, jc]],
                                  ssems[buf]).wait()

    def step(bi, cur, nxt):
        @pl.when(bi >= 1)
        def _():
            drain_scatter(nxt)

        @pl.when(bi + 1 < nb)
        def _():
            stage_idx(bi + 1, nxt)

        wait_idx(cur)
        gather_rows(cur)
        fire_scatter(cur)

    stage_idx(0, 0)

    def body(p, carry):
        bi = p * 2
        step(bi, 0, 1)
        step(bi + 1, 1, 0)
        return carry

    lax.fori_loop(0, nb // 2, body, 0)

    @pl.when(nb % 2 == 1)
    def _():
        step(nb - 1, 0, 1)

    @pl.when(nb % 2 == 1)
    def _():
        drain_scatter(0)

    @pl.when(nb % 2 == 0)
    def _():
        drain_scatter(1)

    plsc.subcore_barrier()

    pltpu.sync_copy(acc_sh.at[sl], bounce)
    pltpu.sync_copy(bounce, part.at[pl.ds(c * ACC_PAD + s * SLICE, SLICE)])


CB = 4000
NT_B = N_NODES // CB


@functools.partial(
    pl.kernel,
    out_type=jax.ShapeDtypeStruct((N_NODES,), jnp.float32),
    mesh=_mesh,
    scratch_types=[
        pltpu.VMEM((CB,), jnp.float32),
        pltpu.VMEM((CB,), jnp.float32),
        pltpu.VMEM((CB,), jnp.float32),
        pltpu.VMEM((CB,), jnp.float32),
        pltpu.VMEM((16,), jnp.float32),
        pltpu.VMEM((16,), jnp.float32),
        pltpu.VMEM((16,), jnp.float32),
    ],
)
def _combine_kernel(part, state, w0, w1, bb, out,
                    p0v, p1v, sv, ov, w0v, w1v, bbv):
    c = lax.axis_index("c")
    s = lax.axis_index("s")
    w = s * NC + c

    @pl.when(w < NT_B)
    def _():
        base = w * CB
        pltpu.sync_copy(part.at[pl.ds(base, CB)], p0v)
        pltpu.sync_copy(part.at[pl.ds(ACC_PAD + base, CB)], p1v)
        pltpu.sync_copy(state.at[pl.ds(base, CB)], sv)
        pltpu.sync_copy(w0, w0v)
        pltpu.sync_copy(w1, w1v)
        pltpu.sync_copy(bb, bbv)
        a0 = w0v[...]
        a1 = w1v[...]
        ab = bbv[...]

        def body(i, carry):
            ch = pl.ds(i * 16, 16)
            ov[ch] = sv[ch] * a0 + (p0v[ch] + p1v[ch]) * a1 + ab
            return carry

        lax.fori_loop(0, CB // 16, body, 0)
        pltpu.sync_copy(ov, out.at[pl.ds(base, CB)])


def kernel(state, edge_index, W, b):
    if state.ndim == 1:
        state = state[:, None]
    sflat = state.reshape(-1).astype(jnp.float32)
    spad = jnp.pad(sflat, (0, ACC_PAD - N_NODES))
    ei = edge_index.astype(jnp.int32)
    part = _scatter_kernel(ei, spad.reshape(ST_ROWS, LANES))
    w0 = jnp.full((16,), W[0, 0], jnp.float32)
    w1 = jnp.full((16,), W[0, 1], jnp.float32)
    bb = jnp.full((16,), b[0], jnp.float32)
    out = _combine_kernel(part, sflat, w0, w1, bb)
    return out.reshape(N_NODES, 1)


# speedup vs baseline: 1.1658x; 1.0024x over previous
import functools

import jax
import jax.numpy as jnp
from jax import lax
from jax.experimental import pallas as pl
from jax.experimental.pallas import tpu as pltpu
from jax.experimental.pallas import tpu_sc as plsc

N_NODES = 100000
N_EDGES = 6400000
LANES = 128
KB = 16
R = N_EDGES // LANES
NB = R // KB
NC, NS = 2, 16
NW = NC * NS
ACC_PAD = 100096
SLICE = ACC_PAD // NS
ST_ROWS = ACC_PAD // LANES

_mesh = plsc.VectorSubcoreMesh(core_axis_name="c", subcore_axis_name="s")


@functools.partial(
    pl.kernel,
    out_type=jax.ShapeDtypeStruct((NC * ACC_PAD,), jnp.float32),
    mesh=_mesh,
    compiler_params=pltpu.CompilerParams(needs_layout_passes=False),
    scratch_types=[
        pltpu.VMEM((2, KB * LANES), jnp.int32),
        pltpu.VMEM((2, KB * LANES), jnp.int32),
        pltpu.VMEM((2, KB, LANES), jnp.float32),
        pltpu.VMEM((SLICE,), jnp.float32),
        pltpu.VMEM((N_NODES,), jnp.float32),
        pltpu.VMEM_SHARED((ACC_PAD,), jnp.float32),
        pltpu.SemaphoreType.DMA,
        pltpu.SemaphoreType.DMA,
        pltpu.SemaphoreType.DMA,
        pltpu.SemaphoreType.DMA,
    ],
)
def _scatter_kernel(ei, state, part, src_i, dst_i, vals, bounce, state_t,
                    acc_sh, isem_a, isem_b, ssem_a, ssem_b):
    c = lax.axis_index("c")
    s = lax.axis_index("s")
    w = s * NC + c

    def zbody(i, carry):
        bounce[pl.ds(i * 16, 16)] = jnp.zeros((16,), jnp.float32)
        return carry

    lax.fori_loop(0, SLICE // 16, zbody, 0)
    sl = pl.ds(s * SLICE, SLICE)
    pltpu.sync_copy(bounce, acc_sh.at[sl])
    pltpu.sync_copy(state, state_t)
    plsc.subcore_barrier()

    nb = (NB - w + NW - 1) // NW
    isems = (isem_a, isem_b)
    ssems = (ssem_a, ssem_b)

    def stage_idx(bi, buf):
        row0 = (w + bi * NW) * (KB * LANES)
        pltpu.async_copy(ei.at[0, pl.ds(row0, KB * LANES)], src_i.at[buf],
                         isems[buf])
        pltpu.async_copy(ei.at[1, pl.ds(row0, KB * LANES)], dst_i.at[buf],
                         isems[buf])

    def wait_idx(buf):
        pltpu.make_async_copy(ei.at[0, pl.ds(0, KB * LANES)], src_i.at[buf],
                              isems[buf]).wait()
        pltpu.make_async_copy(ei.at[1, pl.ds(0, KB * LANES)], dst_i.at[buf],
                              isems[buf]).wait()

    def gather_rows(buf):
        for j in range(KB):
            for k in range(LANES // 16):
                iv = src_i[buf, pl.ds(j * LANES + k * 16, 16)]
                vals[buf, j, pl.ds(k * 16, 16)] = plsc.load_gather(
                    state_t, [iv])

    def fire_scatter(buf):
        for j in range(KB):
            jc = pl.ds(j * LANES, LANES)
            pltpu.async_copy(vals.at[buf, j], acc_sh.at[dst_i.at[buf, jc]],
                             ssems[buf], add=True)

    def drain_scatter(buf):
        for j in range(KB):
            jc = pl.ds(j * LANES, LANES)
            pltpu.make_async_copy(vals.at[buf, j],
                                  acc_sh.at[dst_i.at[buf, jc]],
                                  ssems[buf]).wait()

    def step(bi, cur, nxt):
        @pl.when(bi >= 1)
        def _():
            drain_scatter(nxt)

        @pl.when(bi + 1 < nb)
        def _():
            stage_idx(bi + 1, nxt)

        wait_idx(cur)
        gather_rows(cur)
        fire_scatter(cur)

    stage_idx(0, 0)

    def body(p, carry):
        bi = p * 2
        step(bi, 0, 1)
        step(bi + 1, 1, 0)
        return carry

    lax.fori_loop(0, nb // 2, body, 0)

    @pl.when(nb % 2 == 1)
    def _():
        step(nb - 1, 0, 1)

    @pl.when(nb % 2 == 1)
    def _():
        drain_scatter(0)

    @pl.when(nb % 2 == 0)
    def _():
        drain_scatter(1)

    plsc.subcore_barrier()

    pltpu.sync_copy(acc_sh.at[sl], bounce)
    pltpu.sync_copy(bounce, part.at[pl.ds(c * ACC_PAD + s * SLICE, SLICE)])


CB = 4000
NT_B = N_NODES // CB


@functools.partial(
    pl.kernel,
    out_type=jax.ShapeDtypeStruct((N_NODES,), jnp.float32),
    mesh=_mesh,
    scratch_types=[
        pltpu.VMEM((CB,), jnp.float32),
        pltpu.VMEM((CB,), jnp.float32),
        pltpu.VMEM((CB,), jnp.float32),
        pltpu.VMEM((CB,), jnp.float32),
        pltpu.VMEM((16,), jnp.float32),
        pltpu.VMEM((16,), jnp.float32),
        pltpu.VMEM((16,), jnp.float32),
    ],
)
def _combine_kernel(part, state, w0, w1, bb, out,
                    p0v, p1v, sv, ov, w0v, w1v, bbv):
    c = lax.axis_index("c")
    s = lax.axis_index("s")
    w = s * NC + c

    @pl.when(w < NT_B)
    def _():
        base = w * CB
        pltpu.sync_copy(part.at[pl.ds(base, CB)], p0v)
        pltpu.sync_copy(part.at[pl.ds(ACC_PAD + base, CB)], p1v)
        pltpu.sync_copy(state.at[pl.ds(base, CB)], sv)
        pltpu.sync_copy(w0, w0v)
        pltpu.sync_copy(w1, w1v)
        pltpu.sync_copy(bb, bbv)
        a0 = w0v[...]
        a1 = w1v[...]
        ab = bbv[...]

        def body(i, carry):
            ch = pl.ds(i * 16, 16)
            ov[ch] = sv[ch] * a0 + (p0v[ch] + p1v[ch]) * a1 + ab
            return carry

        lax.fori_loop(0, CB // 16, body, 0)
        pltpu.sync_copy(ov, out.at[pl.ds(base, CB)])


def kernel(state, edge_index, W, b):
    if state.ndim == 1:
        state = state[:, None]
    sflat = state.reshape(-1).astype(jnp.float32)
    ei = edge_index.astype(jnp.int32)
    part = _scatter_kernel(ei, sflat)
    w0 = jnp.full((16,), W[0, 0], jnp.float32)
    w1 = jnp.full((16,), W[0, 1], jnp.float32)
    bb = jnp.full((16,), b[0], jnp.float32)
    out = _combine_kernel(part, sflat, w0, w1, bb)
    return out.reshape(N_NODES, 1)
